# Initial kernel scaffold; baseline (speedup 1.0000x reference)
import jax
import jax.numpy as jnp
from jax.experimental import pallas as pl


def kernel(gene_idx, disease_idx, gene_feat, disease_feat, gene_table, disease_table, Wg, bg, Wd, bd, W1, b1, W2, b2, Wout, bout):
    raise NotImplementedError("write your pallas kernel here")


# speedup vs baseline: 1.3602x; 1.3602x over previous
import functools

import jax
import jax.numpy as jnp
from jax import lax
from jax.experimental import pallas as pl
from jax.experimental.pallas import tpu as pltpu
from jax.experimental.pallas import tpu_sc as plsc

_B = 16384
_E = 32
_F = 64
_KC = 128


def _sc_gather_fn():
    info = plsc.get_sparse_core_info()
    nc, ns = info.num_cores, info.num_subcores
    nw = nc * ns
    bpw = _B // nw
    nchunk = bpw // _KC

    mesh = plsc.VectorSubcoreMesh(core_axis_name="c", subcore_axis_name="s")

    @functools.partial(
        pl.kernel,
        mesh=mesh,
        compiler_params=pltpu.CompilerParams(use_tc_tiling_on_sc=False),
        out_type=[
            jax.ShapeDtypeStruct((_B, _E), jnp.float32),
            jax.ShapeDtypeStruct((_B, _E), jnp.float32),
        ],
        scratch_types=[
            pltpu.VMEM((nchunk, _KC), jnp.int32),
            pltpu.VMEM((nchunk, _KC), jnp.int32),
            pltpu.VMEM((bpw, _E), jnp.float32),
            pltpu.VMEM((bpw, _E), jnp.float32),
            pltpu.SemaphoreType.DMA,
            pltpu.SemaphoreType.DMA,
        ],
    )
    def sc_gather(gidx_hbm, didx_hbm, gtab_hbm, dtab_hbm, ge_hbm, de_hbm,
                  gidx_v, didx_v, grow_v, drow_v, gsem, dsem):
        wid = lax.axis_index("s") * nc + lax.axis_index("c")
        base = wid * bpw
        pltpu.sync_copy(gidx_hbm.at[wid], gidx_v)
        pltpu.sync_copy(didx_hbm.at[wid], didx_v)
        copies = []
        for j in range(nchunk):
            copies.append(pltpu.async_copy(
                gtab_hbm.at[gidx_v.at[j]], grow_v.at[pl.ds(j * _KC, _KC)], gsem))
            copies.append(pltpu.async_copy(
                dtab_hbm.at[didx_v.at[j]], drow_v.at[pl.ds(j * _KC, _KC)], dsem))
        for c in copies:
            c.wait()
        pltpu.sync_copy(grow_v, ge_hbm.at[pl.ds(base, bpw)])
        pltpu.sync_copy(drow_v, de_hbm.at[pl.ds(base, bpw)])

    return sc_gather, nw, nchunk


def _mlp_body(ge_ref, de_ref, gf_ref, df_ref, wg_ref, bg_ref, wd_ref, bd_ref,
              w1_ref, b1_ref, w2_ref, b2_ref, wout_ref, bout_ref, out_ref):
    cdims = (((1,), (1,)), ((), ()))
    sg = lax.dot_general(gf_ref[...], wg_ref[...], cdims,
                         preferred_element_type=jnp.float32) + bg_ref[...]
    sd = lax.dot_general(df_ref[...], wd_ref[...], cdims,
                         preferred_element_type=jnp.float32) + bd_ref[...]
    x = jnp.concatenate([ge_ref[...], de_ref[...], sg, sd], axis=1)
    h1 = jnp.maximum(lax.dot_general(x, w1_ref[...], cdims,
                                     preferred_element_type=jnp.float32)
                     + b1_ref[...], 0.0)
    h2 = jnp.maximum(lax.dot_general(h1, w2_ref[...], cdims,
                                     preferred_element_type=jnp.float32)
                     + b2_ref[...], 0.0)
    out = jnp.sum(h2 * wout_ref[...][None, :], axis=1) + bout_ref[...]
    out_ref[...] = out


def _mlp_call(ge, de, gf, df, wg, bg, wd, bd, w1, b1, w2, b2, wout, bout):
    blk = 2048
    grid = (_B // blk,)
    full2 = lambda shape: pl.BlockSpec(shape, lambda i: (0, 0))
    full1 = lambda shape: pl.BlockSpec(shape, lambda i: (0,))
    return pl.pallas_call(
        _mlp_body,
        grid=grid,
        in_specs=[
            pl.BlockSpec((blk, _E), lambda i: (i, 0)),
            pl.BlockSpec((blk, _E), lambda i: (i, 0)),
            pl.BlockSpec((blk, _F), lambda i: (i, 0)),
            pl.BlockSpec((blk, _F), lambda i: (i, 0)),
            full2((_E, _F)),
            full1((_E,)),
            full2((_E, _F)),
            full1((_E,)),
            full2((128, 4 * _E)),
            full1((128,)),
            full2((64, 128)),
            full1((64,)),
            full1((64,)),
            full1((1,)),
        ],
        out_specs=pl.BlockSpec((blk,), lambda i: (i,)),
        out_shape=jax.ShapeDtypeStruct((_B,), jnp.float32),
    )(ge, de, gf, df, wg, bg, wd, bd, w1, b1, w2, b2, wout, bout)


def kernel(gene_idx, disease_idx, gene_feat, disease_feat, gene_table,
           disease_table, Wg, bg, Wd, bd, W1, b1, W2, b2, Wout, bout):
    sc_gather, nw, nchunk = _sc_gather_fn()
    gidx = gene_idx.astype(jnp.int32).reshape(nw, nchunk, _KC)
    didx = disease_idx.astype(jnp.int32).reshape(nw, nchunk, _KC)
    ge, de = sc_gather(gidx, didx, gene_table, disease_table)
    return _mlp_call(ge, de, gene_feat, disease_feat, Wg, bg, Wd, bd,
                     W1, b1, W2, b2, Wout.reshape(-1), bout)
